# baseline (speedup 1.0000x reference)
import functools

import jax
import jax.numpy as jnp
from jax import lax
from jax.experimental import pallas as pl
from jax.experimental.pallas import tpu as pltpu
from jax.experimental.pallas import tpu_sc as plsc


def _round_bf16(x):
    b = jax.lax.bitcast_convert_type(x, jnp.uint32)
    lsb = (b >> 16) & jnp.uint32(1)
    b = (b + jnp.uint32(0x7FFF) + lsb) & jnp.uint32(0xFFFF0000)
    return jax.lax.bitcast_convert_type(b, jnp.float32)


B, C, H, W = 16, 96, 32, 32
DIM, K = 32, 8192
N = B * H * W

TN = 256
NT = N // TN
SEG = 4096
CHW = 128


def _tc_body(xp_ref, e_ref, fsq_ref, ind_ref, dsum_ref):
    i = pl.program_id(0)
    xp = xp_ref[...]
    emb = e_ref[...]

    fsq = fsq_ref[...]
    esq = jnp.sum(emb ** 2, axis=0, keepdims=True)
    xp2 = xp + xp
    scores2 = jnp.dot(xp2, emb, preferred_element_type=jnp.float32)
    dist = fsq - scores2 + esq

    lane = lax.broadcasted_iota(jnp.int32, (TN, CHW), 1)
    run_bf = None
    for s in range(K // SEG):
        rm = dist[:, s * SEG:s * SEG + CHW]
        rc = jnp.zeros((TN, CHW), jnp.int32)
        for c in range(1, SEG // CHW):
            d_c = dist[:, s * SEG + c * CHW:s * SEG + (c + 1) * CHW]
            lt = d_c < rm
            rm = jnp.where(lt, d_c, rm)
            rc = jnp.where(lt, c, rc)
        mv = jnp.min(rm, axis=1, keepdims=True)
        kf = rc * CHW + lane + (s * SEG)
        li = jnp.min(jnp.where(rm == mv, kf, K),
                     axis=1, keepdims=True)
        if run_bf is None:
            run_bf = _round_bf16(mv)
            run_i = li
            run_exact = mv
        else:
            take = mv < run_bf
            run_bf = jnp.where(take, _round_bf16(mv), run_bf)
            run_i = jnp.where(take, li, run_i)
            run_exact = jnp.where(take, mv, run_exact)
    ind = run_i[:, 0]
    ind_ref[...] = ind.reshape(1, 1, TN)

    @pl.when(i == 0)
    def _init():
        dsum_ref[...] = jnp.zeros((1, 1), jnp.float32)

    dsum_ref[...] = dsum_ref[...] + jnp.sum(run_exact).reshape(1, 1)

    @pl.when(i == NT - 1)
    def _finish():
        dsum_ref[...] = dsum_ref[...] * (1.0 / (N * DIM))


_tc_call = pl.pallas_call(
    _tc_body,
    grid=(NT,),
    in_specs=[
        pl.BlockSpec((TN, DIM), lambda i: (i, 0)),
        pl.BlockSpec((DIM, K), lambda i: (0, 0)),
        pl.BlockSpec((TN, 1), lambda i: (i, 0)),
    ],
    out_specs=[
        pl.BlockSpec((1, 1, TN), lambda i: (i, 0, 0)),
        pl.BlockSpec((1, 1), lambda i: (0, 0)),
    ],
    out_shape=[
        jax.ShapeDtypeStruct((NT, 1, TN), jnp.int32),
        jax.ShapeDtypeStruct((1, 1), jnp.float32),
    ],
)


_NC, _NS = 2, 16
_NW = _NC * _NS
_BPW = N // _NW
_CHUNK = 128
_NCHUNK = _BPW // _CHUNK

@functools.cache
def _make_sc_gather():
    mesh = plsc.VectorSubcoreMesh(core_axis_name="c", subcore_axis_name="s")

    @functools.partial(
        pl.kernel,
        mesh=mesh,
        compiler_params=pltpu.CompilerParams(use_tc_tiling_on_sc=False),
        out_type=jax.ShapeDtypeStruct((N, DIM), jnp.float32),
        scratch_types=[
            pltpu.VMEM((_NCHUNK, _CHUNK), jnp.int32),
            pltpu.VMEM((_BPW, DIM), jnp.float32),
            pltpu.SemaphoreType.DMA,
        ],
    )
    def _sc_gather(table_hbm, idx_hbm, out_hbm, idx_v, rows_v, sem):
        wid = lax.axis_index("s") * _NC + lax.axis_index("c")
        pltpu.sync_copy(idx_hbm.at[pl.ds(wid * _NCHUNK, _NCHUNK)], idx_v)
        for j in range(_NCHUNK):
            pltpu.async_copy(
                table_hbm.at[idx_v.at[j]],
                rows_v.at[pl.ds(j * _CHUNK, _CHUNK)],
                sem,
            ).wait()
        pltpu.sync_copy(rows_v, out_hbm.at[pl.ds(wid * _BPW, _BPW)])

    return _sc_gather


def kernel(x, conv_w, conv_b, embed):
    xp_o = jnp.einsum('bchw,ec->bhwe', x.astype(jnp.float32), conv_w) + conv_b
    flat = xp_o.reshape(-1, DIM)
    fsq = (flat ** 2).sum(1, keepdims=True)
    ind3, dsum = _tc_call(flat, embed, fsq)
    ind = ind3.reshape(N)
    q = _make_sc_gather()(embed.T, ind.reshape(N // _CHUNK, _CHUNK))
    quantize = q.reshape(B, H, W, DIM).transpose(0, 3, 1, 2)
    diff = dsum[0, 0]
    embed_ind_out = ind.reshape(B, H, W)
    return (quantize, diff, embed_ind_out)


# speedup vs baseline: 1.0270x; 1.0270x over previous
import functools

import jax
import jax.numpy as jnp
from jax import lax
from jax.experimental import pallas as pl
from jax.experimental.pallas import tpu as pltpu
from jax.experimental.pallas import tpu_sc as plsc


def _round_bf16(x):
    b = jax.lax.bitcast_convert_type(x, jnp.uint32)
    lsb = (b >> 16) & jnp.uint32(1)
    b = (b + jnp.uint32(0x7FFF) + lsb) & jnp.uint32(0xFFFF0000)
    return jax.lax.bitcast_convert_type(b, jnp.float32)


B, C, H, W = 16, 96, 32, 32
DIM, K = 32, 8192
N = B * H * W

TN = 256
NT = N // TN
SEG = 4096
CHW = 128


def _tc_body(xp_ref, e_ref, fsq_ref, esq_ref, ind_ref, dsum_ref):
    i = pl.program_id(0)
    xp = xp_ref[...]
    emb = e_ref[...]

    fsq = fsq_ref[...]
    esq = esq_ref[...]
    xp2 = xp + xp
    scores2 = jnp.dot(xp2, emb, preferred_element_type=jnp.float32)
    dist = fsq - scores2 + esq

    lane = lax.broadcasted_iota(jnp.int32, (TN, CHW), 1)
    run_bf = None
    for s in range(K // SEG):
        rm = dist[:, s * SEG:s * SEG + CHW]
        rc = jnp.zeros((TN, CHW), jnp.int32)
        for c in range(1, SEG // CHW):
            d_c = dist[:, s * SEG + c * CHW:s * SEG + (c + 1) * CHW]
            lt = d_c < rm
            rm = jnp.where(lt, d_c, rm)
            rc = jnp.where(lt, c, rc)
        mv = jnp.min(rm, axis=1, keepdims=True)
        kf = rc * CHW + lane + (s * SEG)
        li = jnp.min(jnp.where(rm == mv, kf, K),
                     axis=1, keepdims=True)
        if run_bf is None:
            run_bf = _round_bf16(mv)
            run_i = li
            run_exact = mv
        else:
            take = mv < run_bf
            run_bf = jnp.where(take, _round_bf16(mv), run_bf)
            run_i = jnp.where(take, li, run_i)
            run_exact = jnp.where(take, mv, run_exact)
    ind = run_i[:, 0]
    ind_ref[...] = ind.reshape(1, 1, TN)

    @pl.when(i == 0)
    def _init():
        dsum_ref[...] = jnp.zeros((1, 1), jnp.float32)

    dsum_ref[...] = dsum_ref[...] + jnp.sum(run_exact).reshape(1, 1)

    @pl.when(i == NT - 1)
    def _finish():
        dsum_ref[...] = dsum_ref[...] * (1.0 / (N * DIM))


_tc_call = pl.pallas_call(
    _tc_body,
    grid=(NT,),
    in_specs=[
        pl.BlockSpec((TN, DIM), lambda i: (i, 0)),
        pl.BlockSpec((DIM, K), lambda i: (0, 0)),
        pl.BlockSpec((TN, 1), lambda i: (i, 0)),
        pl.BlockSpec((1, K), lambda i: (0, 0)),
    ],
    out_specs=[
        pl.BlockSpec((1, 1, TN), lambda i: (i, 0, 0)),
        pl.BlockSpec((1, 1), lambda i: (0, 0)),
    ],
    out_shape=[
        jax.ShapeDtypeStruct((NT, 1, TN), jnp.int32),
        jax.ShapeDtypeStruct((1, 1), jnp.float32),
    ],
)


_NC, _NS = 2, 16
_NW = _NC * _NS
_BPW = N // _NW
_CHUNK = 128
_NCHUNK = _BPW // _CHUNK

@functools.cache
def _make_sc_gather():
    mesh = plsc.VectorSubcoreMesh(core_axis_name="c", subcore_axis_name="s")

    @functools.partial(
        pl.kernel,
        mesh=mesh,
        compiler_params=pltpu.CompilerParams(use_tc_tiling_on_sc=False),
        out_type=jax.ShapeDtypeStruct((N, DIM), jnp.float32),
        scratch_types=[
            pltpu.VMEM((_NCHUNK, _CHUNK), jnp.int32),
            pltpu.VMEM((_BPW, DIM), jnp.float32),
            pltpu.SemaphoreType.DMA,
        ],
    )
    def _sc_gather(table_hbm, idx_hbm, out_hbm, idx_v, rows_v, sem):
        wid = lax.axis_index("s") * _NC + lax.axis_index("c")
        pltpu.sync_copy(idx_hbm.at[pl.ds(wid * _NCHUNK, _NCHUNK)], idx_v)
        for j in range(_NCHUNK):
            pltpu.async_copy(
                table_hbm.at[idx_v.at[j]],
                rows_v.at[pl.ds(j * _CHUNK, _CHUNK)],
                sem,
            ).wait()
        pltpu.sync_copy(rows_v, out_hbm.at[pl.ds(wid * _BPW, _BPW)])

    return _sc_gather


def kernel(x, conv_w, conv_b, embed):
    xp_o = jnp.einsum('bchw,ec->bhwe', x.astype(jnp.float32), conv_w) + conv_b
    flat = xp_o.reshape(-1, DIM)
    fsq = (flat ** 2).sum(1, keepdims=True)
    esq = (embed ** 2).sum(0, keepdims=True)
    ind3, dsum = _tc_call(flat, embed, fsq, esq)
    ind = ind3.reshape(N)
    q = _make_sc_gather()(embed.T, ind.reshape(N // _CHUNK, _CHUNK))
    quantize = q.reshape(B, H, W, DIM).transpose(0, 3, 1, 2)
    diff = dsum[0, 0]
    embed_ind_out = ind.reshape(B, H, W)
    return (quantize, diff, embed_ind_out)
